# baseline (speedup 1.0000x reference)
import jax
import jax.numpy as jnp
from jax import lax
from jax.experimental import pallas as pl
from jax.experimental.pallas import tpu as pltpu
from jax.experimental.pallas import tpu_sc as plsc

N = 10000
D = 128
E = 320000

NC = 2
NS = 16
NW = NC * NS

C = 128
CHUNKS = 80
E_PAD = NW * CHUNKS * C
DUMMY = N

CCNT = 256
CHUNKS_CNT = E_PAD // (NW * CCNT)

NPAD = 10240
RPT = NPAD // NS
ROW_BLKS = RPT // C

_MESH = plsc.VectorSubcoreMesh(core_axis_name="c", subcore_axis_name="s")


def _agg_body(x_hbm, src_hbm, dst_hbm, za_hbm, agg_out,
              sidx0, didx0, sidx1, didx1, rows0, rows1,
              gsem0, gsem1, ssem0, ssem1, agg_sh):
    c = lax.axis_index("c")
    s = lax.axis_index("s")
    wid = s * NC + c
    base = s * RPT
    sidx = (sidx0, sidx1)
    didx = (didx0, didx1)
    rows = (rows0, rows1)
    gsem = (gsem0, gsem1)
    ssem = (ssem0, ssem1)

    pltpu.sync_copy(za_hbm, rows0)
    for k in range(ROW_BLKS):
        pltpu.sync_copy(rows0, agg_sh.at[pl.ds(base + k * C, C)])
    plsc.subcore_barrier()

    def load_idx(j, b):
        pltpu.sync_copy(src_hbm.at[wid, j], sidx[b])
        pltpu.sync_copy(dst_hbm.at[wid, j], didx[b])

    def start_g(b):
        pltpu.async_copy(x_hbm.at[sidx[b]], rows[b], gsem[b])

    def wait_g(b):
        pltpu.make_async_copy(x_hbm.at[sidx[b]], rows[b], gsem[b]).wait()

    def start_s(b):
        pltpu.async_copy(rows[b], agg_sh.at[didx[b]], ssem[b], add=True)

    def wait_s(b):
        pltpu.make_async_copy(rows[b], agg_sh.at[didx[b]], ssem[b]).wait()

    load_idx(0, 0)
    start_g(0)
    wait_g(0)
    start_s(0)
    load_idx(1, 1)
    start_g(1)

    def main(t, carry):
        j = 1 + 2 * t
        wait_g(1)
        start_s(1)
        wait_s(0)
        load_idx(j + 1, 0)
        start_g(0)
        wait_g(0)
        start_s(0)
        wait_s(1)
        load_idx(j + 2, 1)
        start_g(1)
        return carry

    lax.fori_loop(0, (CHUNKS - 2) // 2, main, 0)
    wait_g(1)
    start_s(1)
    wait_s(0)
    wait_s(1)
    plsc.subcore_barrier()

    for k in range(ROW_BLKS):
        pltpu.sync_copy(agg_sh.at[pl.ds(base + k * C, C)], rows0)
        pltpu.sync_copy(rows0, agg_out.at[c, pl.ds(base + k * C, C)])


def _cnt_body(dst_hbm, zc_hbm, on_hbm, cnt_out,
              didx0, didx1, rows, ssem0, ssem1, cnt_sh):
    c = lax.axis_index("c")
    s = lax.axis_index("s")
    wid = s * NC + c
    base = s * RPT
    didx = (didx0, didx1)
    ssem = (ssem0, ssem1)

    pltpu.sync_copy(zc_hbm, rows)
    for k in range(ROW_BLKS):
        pltpu.sync_copy(rows.at[pl.ds(0, C)], cnt_sh.at[pl.ds(base + k * C, C)])
    pltpu.sync_copy(on_hbm, rows)
    plsc.subcore_barrier()

    def load_idx(j, b):
        pltpu.sync_copy(dst_hbm.at[wid, j], didx[b])

    def start_s(b):
        pltpu.async_copy(rows, cnt_sh.at[didx[b]], ssem[b], add=True)

    def wait_s(b):
        pltpu.make_async_copy(rows, cnt_sh.at[didx[b]], ssem[b]).wait()

    load_idx(0, 0)
    start_s(0)
    load_idx(1, 1)
    start_s(1)

    def main(t, carry):
        j = 2 * t
        wait_s(0)
        load_idx(j, 0)
        start_s(0)
        wait_s(1)
        load_idx(j + 1, 1)
        start_s(1)
        return carry

    lax.fori_loop(1, CHUNKS_CNT // 2, main, 0)
    wait_s(0)
    wait_s(1)
    plsc.subcore_barrier()

    for k in range(ROW_BLKS):
        pltpu.sync_copy(cnt_sh.at[pl.ds(base + k * C, C)], rows.at[pl.ds(0, C)])
        pltpu.sync_copy(rows.at[pl.ds(0, C)], cnt_out.at[c, pl.ds(base + k * C, C)])


_sc_agg = pl.kernel(
    _agg_body,
    out_type=jax.ShapeDtypeStruct((NC, NPAD, D), jnp.float32),
    mesh=_MESH,
    scratch_types=[
        pltpu.VMEM((C,), jnp.int32),
        pltpu.VMEM((C,), jnp.int32),
        pltpu.VMEM((C,), jnp.int32),
        pltpu.VMEM((C,), jnp.int32),
        pltpu.VMEM((C, D), jnp.float32),
        pltpu.VMEM((C, D), jnp.float32),
        pltpu.SemaphoreType.DMA,
        pltpu.SemaphoreType.DMA,
        pltpu.SemaphoreType.DMA,
        pltpu.SemaphoreType.DMA,
        pltpu.VMEM_SHARED((NPAD, D), jnp.float32),
    ],
)

_sc_cnt = pl.kernel(
    _cnt_body,
    out_type=jax.ShapeDtypeStruct((NC, NPAD, D), jnp.float32),
    mesh=_MESH,
    scratch_types=[
        pltpu.VMEM((CCNT,), jnp.int32),
        pltpu.VMEM((CCNT,), jnp.int32),
        pltpu.VMEM((CCNT, D), jnp.float32),
        pltpu.SemaphoreType.DMA,
        pltpu.SemaphoreType.DMA,
        pltpu.VMEM_SHARED((NPAD, D), jnp.float32),
    ],
)


def _tc_body(x_ref, agg_ref, cnt_ref, wl_ref, wr_ref, b_ref, o_ref):
    a = agg_ref[0] + agg_ref[1]
    cn = cnt_ref[0, :, 0:1] + cnt_ref[1, :, 0:1]
    inv = 1.0 / jnp.maximum(cn, 1.0)
    am = a * inv
    acc = lax.dot_general(am, wl_ref[...], (((1,), (1,)), ((), ())),
                          preferred_element_type=jnp.float32)
    acc = acc + lax.dot_general(x_ref[...], wr_ref[...],
                                (((1,), (1,)), ((), ())),
                                preferred_element_type=jnp.float32)
    o_ref[...] = jnp.maximum(acc + b_ref[...], 0.0)


_TC_R = 1000


def _tc_layer(x, agg, cnt, Wl, Wr, b2d):
    grid = N // _TC_R
    return pl.pallas_call(
        _tc_body,
        grid=(grid,),
        in_specs=[
            pl.BlockSpec((_TC_R, D), lambda i: (i, 0)),
            pl.BlockSpec((NC, _TC_R, D), lambda i: (0, i, 0)),
            pl.BlockSpec((NC, _TC_R, D), lambda i: (0, i, 0)),
            pl.BlockSpec((D, D), lambda i: (0, 0)),
            pl.BlockSpec((D, D), lambda i: (0, 0)),
            pl.BlockSpec((1, D), lambda i: (0, 0)),
        ],
        out_specs=pl.BlockSpec((_TC_R, D), lambda i: (i, 0)),
        out_shape=jax.ShapeDtypeStruct((N, D), jnp.float32),
    )(x, agg, cnt, Wl, Wr, b2d)


@jax.jit
def kernel(x, edge_index, W1_l, b1, W1_r, W2_l, b2, W2_r):
    src = edge_index[0].astype(jnp.int32)
    dst = edge_index[1].astype(jnp.int32)
    pad = E_PAD - E
    src = jnp.concatenate([src, jnp.zeros((pad,), jnp.int32)])
    dst = jnp.concatenate([dst, jnp.full((pad,), DUMMY, jnp.int32)])
    src = src.reshape(NW, CHUNKS, C)
    dstc = dst.reshape(NW, CHUNKS_CNT, CCNT)
    dst = dst.reshape(NW, CHUNKS, C)
    zeros_a = jnp.zeros((C, D), jnp.float32)
    zeros_c = jnp.zeros((CCNT, D), jnp.float32)
    ones_c = jnp.ones((CCNT, D), jnp.float32)

    cnt = _sc_cnt(dstc, zeros_c, ones_c)
    agg1 = _sc_agg(x, src, dst, zeros_a)
    h1 = _tc_layer(x, agg1, cnt, W1_l, W1_r, b1.reshape(1, D))
    agg2 = _sc_agg(h1, src, dst, zeros_a)
    h2 = _tc_layer(h1, agg2, cnt, W2_l, W2_r, b2.reshape(1, D))
    return h2
